# baseline (speedup 1.0000x reference)
import jax
import jax.numpy as jnp
from jax import lax
from jax.experimental import pallas as pl
from jax.experimental.pallas import tpu as pltpu

N = 2048
D = 768
E = 8
K = 2
TM = 384
M_TILES = D // TM


def _moe_kernel(x_ref, wg_ref, bg_ref, we_ref, be_ref, out_ref, scale_ref, xb_ref):
    m = pl.program_id(0)

    @pl.when(m == 0)
    def _():
        xb_ref[...] = x_ref[...].astype(jnp.bfloat16)
        logits = lax.dot_general(
            x_ref[...], wg_ref[...], (((1,), (1,)), ((), ())),
            preferred_element_type=jnp.float32,
        ) + bg_ref[...]
        idx = lax.broadcasted_iota(jnp.int32, logits.shape, 1)
        m1 = jnp.max(logits, axis=1, keepdims=True)
        i1 = jnp.min(jnp.where(logits == m1, idx, E), axis=1, keepdims=True)
        oh1 = idx == i1
        masked = jnp.where(oh1, -jnp.inf, logits)
        m2 = jnp.max(masked, axis=1, keepdims=True)
        i2 = jnp.min(jnp.where(masked == m2, idx, E), axis=1, keepdims=True)
        oh2 = idx == i2
        cnt = jnp.sum(oh1.astype(jnp.float32) + oh2.astype(jnp.float32), axis=0)
        scale_ref[...] = (cnt / float(N * K)).reshape(1, E)

    acc = None
    for ee in range(E):
        sel = lax.broadcasted_iota(jnp.int32, (1, E), 1) == ee
        s = jnp.sum(jnp.where(sel, scale_ref[...], 0.0), axis=(0, 1), keepdims=True)
        wb = (we_ref[ee] * s).astype(jnp.bfloat16)
        sb = be_ref[ee] * s
        z = lax.dot_general(
            xb_ref[...], wb, (((1,), (1,)), ((), ())),
            preferred_element_type=jnp.float32,
        )
        r = jnp.maximum(z + sb, 0.0)
        acc = r if acc is None else acc + r
    out_ref[...] = acc


def kernel(x, Wg, bg, We, be):
    out = pl.pallas_call(
        _moe_kernel,
        grid=(M_TILES,),
        in_specs=[
            pl.BlockSpec((N, D), lambda m: (0, 0)),
            pl.BlockSpec((E, D), lambda m: (0, 0)),
            pl.BlockSpec((1, E), lambda m: (0, 0)),
            pl.BlockSpec((E, TM, D), lambda m: (0, m, 0)),
            pl.BlockSpec((E, 1, TM), lambda m: (0, 0, m)),
        ],
        out_specs=pl.BlockSpec((N, TM), lambda m: (0, m)),
        out_shape=jax.ShapeDtypeStruct((N, D), jnp.float32),
        scratch_shapes=[
            pltpu.VMEM((1, E), jnp.float32),
            pltpu.VMEM((N, D), jnp.bfloat16),
        ],
    )(x, Wg, bg.reshape(1, E), We.reshape(E, D, D), be.reshape(E, 1, D))
    return out


# speedup vs baseline: 1.2216x; 1.2216x over previous
import jax
import jax.numpy as jnp
from jax import lax
from jax.experimental import pallas as pl
from jax.experimental.pallas import tpu as pltpu

N = 2048
D = 768
E = 8
K = 2
TM = 256
M_TILES = D // TM


def _moe_kernel(x_ref, wg_ref, bg_ref, we_ref, be_ref, out_ref, scale_ref, xb_ref):
    m = pl.program_id(0)

    @pl.when(m == 0)
    def _():
        xb_ref[...] = x_ref[...].astype(jnp.bfloat16)
        logits = lax.dot_general(
            x_ref[...], wg_ref[...], (((1,), (1,)), ((), ())),
            preferred_element_type=jnp.float32,
        ) + bg_ref[...]
        idx = lax.broadcasted_iota(jnp.int32, logits.shape, 1)
        m1 = jnp.max(logits, axis=1, keepdims=True)
        i1 = jnp.min(jnp.where(logits == m1, idx, E), axis=1, keepdims=True)
        oh1 = idx == i1
        masked = jnp.where(oh1, -jnp.inf, logits)
        m2 = jnp.max(masked, axis=1, keepdims=True)
        i2 = jnp.min(jnp.where(masked == m2, idx, E), axis=1, keepdims=True)
        oh2 = idx == i2
        cnt = jnp.sum(oh1.astype(jnp.float32) + oh2.astype(jnp.float32), axis=0)
        scale_ref[...] = (cnt / float(N * K)).reshape(1, E)

    acc = None
    for ee in range(E):
        sel = lax.broadcasted_iota(jnp.int32, (1, E), 1) == ee
        s = jnp.sum(jnp.where(sel, scale_ref[...], 0.0), axis=(0, 1), keepdims=True)
        wb = (we_ref[ee] * s).astype(jnp.bfloat16)
        sb = be_ref[ee] * s
        z = lax.dot_general(
            xb_ref[...], wb, (((1,), (1,)), ((), ())),
            preferred_element_type=jnp.float32,
        ).astype(jnp.bfloat16)
        r = jnp.maximum(z + sb.astype(jnp.bfloat16), 0).astype(jnp.float32)
        acc = r if acc is None else acc + r
    out_ref[...] = acc


def kernel(x, Wg, bg, We, be):
    out = pl.pallas_call(
        _moe_kernel,
        grid=(M_TILES,),
        in_specs=[
            pl.BlockSpec((N, D), lambda m: (0, 0)),
            pl.BlockSpec((E, D), lambda m: (0, 0)),
            pl.BlockSpec((1, E), lambda m: (0, 0)),
            pl.BlockSpec((E, TM, D), lambda m: (0, m, 0)),
            pl.BlockSpec((E, 1, TM), lambda m: (0, 0, m)),
        ],
        out_specs=pl.BlockSpec((N, TM), lambda m: (0, m)),
        out_shape=jax.ShapeDtypeStruct((N, D), jnp.float32),
        scratch_shapes=[
            pltpu.VMEM((1, E), jnp.float32),
            pltpu.VMEM((N, D), jnp.bfloat16),
        ],
    )(x, Wg, bg.reshape(1, E), We.reshape(E, D, D), be.reshape(E, 1, D))
    return out
